# baseline (speedup 1.0000x reference)
import functools

import jax
import jax.numpy as jnp
from jax import lax
from jax.experimental import pallas as pl
from jax.experimental.pallas import tpu as pltpu
from jax.experimental.pallas import tpu_sc as plsc

_NC = 2
_NS = 16
_CH = 128
_ZR = 64
_BM = 256
_DEGW = 16


def _round_up(v, m):
    return (v + m - 1) // m * m


def _make_sc_deg(NP, EP):
    n_idx_rows = EP // _CH
    rows_per_worker = n_idx_rows // (_NC * _NS)
    acc_rows_per_sub = NP // _NS
    D = _DEGW
    mesh = plsc.VectorSubcoreMesh(core_axis_name="c", subcore_axis_name="s")

    @functools.partial(
        pl.kernel,
        mesh=mesh,
        out_type=jax.ShapeDtypeStruct((_NC * NP, D), jnp.float32),
        scratch_types=[
            pltpu.VMEM((rows_per_worker, _CH), jnp.int32),
            pltpu.VMEM((_ZR, D), jnp.float32),
            pltpu.VMEM((_CH, D), jnp.float32),
            pltpu.VMEM_SHARED((NP, D), jnp.float32),
            pltpu.SemaphoreType.DMA,
        ],
        compiler_params=pltpu.CompilerParams(use_tc_tiling_on_sc=False),
    )
    def k(dst_hbm, out_hbm, dst_v, zbuf, rbuf, acc, sem):
        c = lax.axis_index("c")
        s = lax.axis_index("s")

        def zstore(i, _):
            r = i // (D // 16)
            col = (i % (D // 16)) * 16
            zbuf[r, pl.ds(col, 16)] = jnp.zeros((16,), jnp.float32)
            return 0
        lax.fori_loop(0, _ZR * D // 16, zstore, 0)
        def ostore(i, _):
            r = i // (D // 16)
            col = (i % (D // 16)) * 16
            rbuf[r, pl.ds(col, 16)] = jnp.ones((16,), jnp.float32)
            return 0
        lax.fori_loop(0, _CH * D // 16, ostore, 0)

        row_base = s * acc_rows_per_sub
        wrow = (c * _NS + s) * rows_per_worker
        pltpu.sync_copy(dst_hbm.at[pl.ds(wrow, rows_per_worker)], dst_v)

        def zdesc(j):
            return pltpu.make_async_copy(
                zbuf, acc.at[pl.ds(row_base + j * _ZR, _ZR)], sem)
        def zcopy(j, _):
            zdesc(j).start()
            return 0
        lax.fori_loop(0, acc_rows_per_sub // _ZR, zcopy, 0)
        def zwait(j, _):
            zdesc(j).wait()
            return 0
        lax.fori_loop(0, acc_rows_per_sub // _ZR, zwait, 0)
        plsc.subcore_barrier()

        def step(j, _):
            pltpu.sync_copy(rbuf, acc.at[dst_v.at[j]], add=True)
            return 0
        lax.fori_loop(0, rows_per_worker, step, 0)
        plsc.subcore_barrier()

        pltpu.sync_copy(acc.at[pl.ds(row_base, acc_rows_per_sub)],
                        out_hbm.at[pl.ds(c * NP + row_base, acc_rows_per_sub)])

    return k


def _make_sc_agg(NP, D, EP):
    n_idx_rows = EP // _CH
    rows_per_worker = n_idx_rows // (_NC * _NS)
    acc_rows_per_sub = NP // _NS
    mesh = plsc.VectorSubcoreMesh(core_axis_name="c", subcore_axis_name="s")

    @functools.partial(
        pl.kernel,
        mesh=mesh,
        out_type=jax.ShapeDtypeStruct((_NC * NP, D), jnp.float32),
        scratch_types=[
            pltpu.VMEM((rows_per_worker, _CH), jnp.int32),
            pltpu.VMEM((rows_per_worker, _CH), jnp.int32),
            pltpu.VMEM((_ZR, D), jnp.float32),
            pltpu.VMEM((_CH, D), jnp.float32),
            pltpu.VMEM((_CH, D), jnp.float32),
            pltpu.VMEM_SHARED((NP, D), jnp.float32),
            pltpu.VMEM_SHARED((NP, D), jnp.float32),
            pltpu.SemaphoreType.DMA,
            pltpu.SemaphoreType.DMA,
            pltpu.SemaphoreType.DMA,
            pltpu.SemaphoreType.DMA,
        ],
        compiler_params=pltpu.CompilerParams(use_tc_tiling_on_sc=False),
    )
    def k(g_hbm, src_hbm, dst_hbm, out_hbm, src_v, dst_v, zbuf, rbuf, rbuf1,
          gsh, acc, sem, sem1, ssem, ssem1):
        c = lax.axis_index("c")
        s = lax.axis_index("s")

        def zstore(i, _):
            r = i // (D // 16)
            col = (i % (D // 16)) * 16
            zbuf[r, pl.ds(col, 16)] = jnp.zeros((16,), jnp.float32)
            return 0
        lax.fori_loop(0, _ZR * D // 16, zstore, 0)

        row_base = s * acc_rows_per_sub
        wrow = (c * _NS + s) * rows_per_worker
        pltpu.async_copy(dst_hbm.at[pl.ds(wrow, rows_per_worker)], dst_v, ssem)
        pltpu.async_copy(src_hbm.at[pl.ds(wrow, rows_per_worker)], src_v,
                         ssem1)
        gstage = pltpu.make_async_copy(
            g_hbm.at[pl.ds(row_base, acc_rows_per_sub)],
            gsh.at[pl.ds(row_base, acc_rows_per_sub)], sem1)
        gstage.start()

        def zdesc(j):
            return pltpu.make_async_copy(
                zbuf, acc.at[pl.ds(row_base + j * _ZR, _ZR)], sem)
        def zcopy(j, _):
            zdesc(j).start()
            return 0
        lax.fori_loop(0, acc_rows_per_sub // _ZR, zcopy, 0)
        def zwait(j, _):
            zdesc(j).wait()
            return 0
        lax.fori_loop(0, acc_rows_per_sub // _ZR, zwait, 0)
        gstage.wait()
        pltpu.make_async_copy(
            dst_hbm.at[pl.ds(wrow, rows_per_worker)], dst_v, ssem).wait()
        pltpu.make_async_copy(
            src_hbm.at[pl.ds(wrow, rows_per_worker)], src_v, ssem1).wait()
        plsc.subcore_barrier()

        def _g(j, buf, sm):
            return pltpu.make_async_copy(gsh.at[src_v.at[j]], buf, sm)
        def _s(j, buf, sm):
            return pltpu.make_async_copy(buf, acc.at[dst_v.at[j]], sm)
        _g(0, rbuf, sem).start()
        if rows_per_worker > 1:
            _g(1, rbuf1, sem1).start()

        def step(j2, _):
            j = 2 * j2
            _g(j, rbuf, sem).wait()
            pltpu.async_copy(rbuf, acc.at[dst_v.at[j]], ssem, add=True)
            @pl.when(j + 1 < rows_per_worker)
            def _():
                _g(j + 1, rbuf1, sem1).wait()
                pltpu.async_copy(rbuf1, acc.at[dst_v.at[j + 1]], ssem1,
                                 add=True)
            _s(j, rbuf, ssem).wait()
            @pl.when(j + 2 < rows_per_worker)
            def _():
                _g(j + 2, rbuf, sem).start()
            @pl.when(j + 1 < rows_per_worker)
            def _():
                _s(j + 1, rbuf1, ssem1).wait()
                @pl.when(j + 3 < rows_per_worker)
                def _():
                    _g(j + 3, rbuf1, sem1).start()
            return 0
        lax.fori_loop(0, (rows_per_worker + 1) // 2, step, 0)
        plsc.subcore_barrier()

        pltpu.sync_copy(acc.at[pl.ds(row_base, acc_rows_per_sub)],
                        out_hbm.at[pl.ds(c * NP + row_base, acc_rows_per_sub)])

    return k


def _make_sc_agg2(NP, D, EP):
    n_idx_rows = EP // _CH
    rows_per_worker = n_idx_rows // (_NC * _NS)
    acc_rows_per_sub = NP // _NS
    mesh = plsc.VectorSubcoreMesh(core_axis_name="c", subcore_axis_name="s")

    @functools.partial(
        pl.kernel,
        mesh=mesh,
        out_type=[jax.ShapeDtypeStruct((_NC * NP, D), jnp.float32),
                  jax.ShapeDtypeStruct((_NC * NP, D), jnp.float32)],
        scratch_types=[
            pltpu.VMEM((rows_per_worker, _CH), jnp.int32),
            pltpu.VMEM((rows_per_worker, _CH), jnp.int32),
            pltpu.VMEM((_ZR, D), jnp.float32),
            pltpu.VMEM((_CH, D), jnp.float32),
            pltpu.VMEM((_CH, D), jnp.float32),
            pltpu.VMEM_SHARED((NP, D), jnp.float32),
            pltpu.VMEM_SHARED((NP, D), jnp.float32),
            pltpu.SemaphoreType.DMA,
            pltpu.SemaphoreType.DMA,
            pltpu.SemaphoreType.DMA,
            pltpu.SemaphoreType.DMA,
        ],
        compiler_params=pltpu.CompilerParams(use_tc_tiling_on_sc=False),
    )
    def k(gl_hbm, gr_hbm, src_hbm, dst_hbm, outl_hbm, outr_hbm, src_v, dst_v,
          zbuf, rbuf, rbuf1, gsh, acc, sem, sem1, ssem, ssem1):
        c = lax.axis_index("c")
        s = lax.axis_index("s")

        def zstore(i, _):
            r = i // (D // 16)
            col = (i % (D // 16)) * 16
            zbuf[r, pl.ds(col, 16)] = jnp.zeros((16,), jnp.float32)
            return 0
        lax.fori_loop(0, _ZR * D // 16, zstore, 0)

        row_base = s * acc_rows_per_sub
        wrow = (c * _NS + s) * rows_per_worker
        pltpu.async_copy(dst_hbm.at[pl.ds(wrow, rows_per_worker)], dst_v, ssem)
        pltpu.async_copy(src_hbm.at[pl.ds(wrow, rows_per_worker)], src_v,
                         ssem1)

        def gstage(g_hbm):
            return pltpu.make_async_copy(
                g_hbm.at[pl.ds(row_base, acc_rows_per_sub)],
                gsh.at[pl.ds(row_base, acc_rows_per_sub)], sem1)

        def zdesc(j):
            return pltpu.make_async_copy(
                zbuf, acc.at[pl.ds(row_base + j * _ZR, _ZR)], sem)

        def zero_acc():
            def zcopy(j, _):
                zdesc(j).start()
                return 0
            lax.fori_loop(0, acc_rows_per_sub // _ZR, zcopy, 0)
            def zwait(j, _):
                zdesc(j).wait()
                return 0
            lax.fori_loop(0, acc_rows_per_sub // _ZR, zwait, 0)

        def edge_pipeline():
            def _g(j, buf, sm):
                return pltpu.make_async_copy(gsh.at[src_v.at[j]], buf, sm)
            def _s(j, buf, sm):
                return pltpu.make_async_copy(buf, acc.at[dst_v.at[j]], sm)
            _g(0, rbuf, sem).start()
            if rows_per_worker > 1:
                _g(1, rbuf1, sem1).start()

            def step(j2, _):
                j = 2 * j2
                _g(j, rbuf, sem).wait()
                pltpu.async_copy(rbuf, acc.at[dst_v.at[j]], ssem, add=True)
                @pl.when(j + 1 < rows_per_worker)
                def _():
                    _g(j + 1, rbuf1, sem1).wait()
                    pltpu.async_copy(rbuf1, acc.at[dst_v.at[j + 1]], ssem1,
                                     add=True)
                _s(j, rbuf, ssem).wait()
                @pl.when(j + 2 < rows_per_worker)
                def _():
                    _g(j + 2, rbuf, sem).start()
                @pl.when(j + 1 < rows_per_worker)
                def _():
                    _s(j + 1, rbuf1, ssem1).wait()
                    @pl.when(j + 3 < rows_per_worker)
                    def _():
                        _g(j + 3, rbuf1, sem1).start()
                return 0
            lax.fori_loop(0, (rows_per_worker + 1) // 2, step, 0)

        def out_desc(out_hbm, sm):
            return pltpu.make_async_copy(
                acc.at[pl.ds(row_base, acc_rows_per_sub)],
                out_hbm.at[pl.ds(c * NP + row_base, acc_rows_per_sub)], sm)

        stage = gstage(gl_hbm)
        stage.start()
        zero_acc()
        stage.wait()
        pltpu.make_async_copy(
            dst_hbm.at[pl.ds(wrow, rows_per_worker)], dst_v, ssem).wait()
        pltpu.make_async_copy(
            src_hbm.at[pl.ds(wrow, rows_per_worker)], src_v, ssem1).wait()
        plsc.subcore_barrier()
        edge_pipeline()
        plsc.subcore_barrier()
        outl = out_desc(outl_hbm, ssem)
        outl.start()

        stage = gstage(gr_hbm)
        stage.start()
        outl.wait()
        zero_acc()
        stage.wait()
        plsc.subcore_barrier()
        edge_pipeline()
        plsc.subcore_barrier()
        out_desc(outr_hbm, ssem).start()
        out_desc(outr_hbm, ssem).wait()

    return k


def _dis(dega_ref, degb_ref):
    deg = dega_ref[...][:, :1] + degb_ref[...][:, :1] + 1.0
    return lax.rsqrt(deg)


def _half_specs(d, nblk):
    return [pl.BlockSpec((_BM, d), lambda i: (i, 0)),
            pl.BlockSpec((_BM, d), lambda i, nb=nblk: (i + nb, 0))]


def _mm1_body(dega_ref, degb_ref, x_ref, w_ref, o1_ref, o2_ref):
    dis = _dis(dega_ref, degb_ref)
    h = w_ref.shape[1] // 2
    res = jnp.dot(x_ref[...], w_ref[...],
                  preferred_element_type=jnp.float32) * dis
    o1_ref[...] = res[:, :h]
    o2_ref[...] = res[:, h:]


def _mid_body(dega_ref, degb_ref, sla_ref, slb_ref, sra_ref, srb_ref,
              g1l_ref, g1r_ref, b1_ref, w2_ref, o_ref):
    dis = _dis(dega_ref, degb_ref)
    h = g1l_ref.shape[1]
    ul = dis * (sla_ref[...] + slb_ref[...] + g1l_ref[...]) + b1_ref[:, :h]
    ur = dis * (sra_ref[...] + srb_ref[...] + g1r_ref[...]) + b1_ref[:, h:]
    u = jnp.maximum(jnp.concatenate([ul, ur], axis=1), 0.0)
    o_ref[...] = jnp.dot(u, w2_ref[...], preferred_element_type=jnp.float32) * dis


def _fin_body(dega_ref, degb_ref, s2a_ref, s2b_ref, g2_ref, b2_ref, o_ref):
    dis = _dis(dega_ref, degb_ref)
    o_ref[...] = dis * (s2a_ref[...] + s2b_ref[...] + g2_ref[...]) + b2_ref[...]


def _row_spec(d):
    return pl.BlockSpec((_BM, d), lambda i: (i, 0))


def _full_spec(r, c):
    return pl.BlockSpec((r, c), lambda i: (0, 0))


def kernel(x, edge_index, W1, b1, W2, b2):
    N, F = x.shape
    E = edge_index.shape[1]
    D1 = W1.shape[1]
    D2 = W2.shape[1]
    DH = D1 // 2
    NP = _round_up(N + 1, _NS * _ZR)
    EP = _round_up(E, _NC * _NS * _CH)

    pad = jnp.full((EP - E,), N, jnp.int32)
    src = jnp.concatenate([edge_index[0], pad]).reshape(EP // _CH, _CH)
    dst = jnp.concatenate([edge_index[1], pad]).reshape(EP // _CH, _CH)

    degp = _make_sc_deg(NP, EP)(dst)

    nblk = NP // _BM
    grid = (nblk,)
    deg_specs = _half_specs(_DEGW, nblk)

    g1l, g1r = pl.pallas_call(
        _mm1_body,
        grid=grid,
        in_specs=deg_specs + [_row_spec(F), _full_spec(F, D1)],
        out_specs=[_row_spec(DH), _row_spec(DH)],
        out_shape=[jax.ShapeDtypeStruct((NP, DH), jnp.float32),
                   jax.ShapeDtypeStruct((NP, DH), jnp.float32)],
    )(degp, degp, x, W1)

    s1l, s1r = _make_sc_agg2(NP, DH, EP)(g1l, g1r, src, dst)

    g2 = pl.pallas_call(
        _mid_body,
        grid=grid,
        in_specs=(deg_specs + _half_specs(DH, nblk) + _half_specs(DH, nblk)
                  + [_row_spec(DH), _row_spec(DH), _full_spec(1, D1),
                     _full_spec(D1, D2)]),
        out_specs=_row_spec(D2),
        out_shape=jax.ShapeDtypeStruct((NP, D2), jnp.float32),
    )(degp, degp, s1l, s1l, s1r, s1r, g1l, g1r, b1.reshape(1, D1), W2)

    s2 = _make_sc_agg(NP, D2, EP)(g2, src, dst)

    out = pl.pallas_call(
        _fin_body,
        grid=grid,
        in_specs=deg_specs + _half_specs(D2, nblk) + [_row_spec(D2),
                  _full_spec(1, D2)],
        out_specs=_row_spec(D2),
        out_shape=jax.ShapeDtypeStruct((N, D2), jnp.float32),
    )(degp, degp, s2, s2, g2, b2.reshape(1, D2))

    return out


# speedup vs baseline: 1.0360x; 1.0360x over previous
import functools

import jax
import jax.numpy as jnp
from jax import lax
from jax.experimental import pallas as pl
from jax.experimental.pallas import tpu as pltpu
from jax.experimental.pallas import tpu_sc as plsc

_NC = 2
_NS = 16
_CH = 128
_ZR = 64
_BM = 256
_DEGW = 16


def _round_up(v, m):
    return (v + m - 1) // m * m


def _make_sc_deg(NP, EP):
    n_idx_rows = EP // _CH
    rows_per_worker = n_idx_rows // (_NC * _NS)
    acc_rows_per_sub = NP // _NS
    D = _DEGW
    mesh = plsc.VectorSubcoreMesh(core_axis_name="c", subcore_axis_name="s")

    @functools.partial(
        pl.kernel,
        mesh=mesh,
        out_type=jax.ShapeDtypeStruct((_NC * NP, D), jnp.float32),
        scratch_types=[
            pltpu.VMEM((rows_per_worker, _CH), jnp.int32),
            pltpu.VMEM((_ZR, D), jnp.float32),
            pltpu.VMEM((_CH, D), jnp.float32),
            pltpu.VMEM_SHARED((NP, D), jnp.float32),
            pltpu.SemaphoreType.DMA,
        ],
        compiler_params=pltpu.CompilerParams(use_tc_tiling_on_sc=False),
    )
    def k(dst_hbm, out_hbm, dst_v, zbuf, rbuf, acc, sem):
        c = lax.axis_index("c")
        s = lax.axis_index("s")

        def zstore(i, _):
            r = i // (D // 16)
            col = (i % (D // 16)) * 16
            zbuf[r, pl.ds(col, 16)] = jnp.zeros((16,), jnp.float32)
            return 0
        lax.fori_loop(0, _ZR * D // 16, zstore, 0)
        def ostore(i, _):
            r = i // (D // 16)
            col = (i % (D // 16)) * 16
            rbuf[r, pl.ds(col, 16)] = jnp.ones((16,), jnp.float32)
            return 0
        lax.fori_loop(0, _CH * D // 16, ostore, 0)

        row_base = s * acc_rows_per_sub
        wrow = (c * _NS + s) * rows_per_worker
        pltpu.sync_copy(dst_hbm.at[pl.ds(wrow, rows_per_worker)], dst_v)

        def zdesc(j):
            return pltpu.make_async_copy(
                zbuf, acc.at[pl.ds(row_base + j * _ZR, _ZR)], sem)
        def zcopy(j, _):
            zdesc(j).start()
            return 0
        lax.fori_loop(0, acc_rows_per_sub // _ZR, zcopy, 0)
        def zwait(j, _):
            zdesc(j).wait()
            return 0
        lax.fori_loop(0, acc_rows_per_sub // _ZR, zwait, 0)
        plsc.subcore_barrier()

        def step(j, _):
            pltpu.sync_copy(rbuf, acc.at[dst_v.at[j]], add=True)
            return 0
        lax.fori_loop(0, rows_per_worker, step, 0)
        plsc.subcore_barrier()

        pltpu.sync_copy(acc.at[pl.ds(row_base, acc_rows_per_sub)],
                        out_hbm.at[pl.ds(c * NP + row_base, acc_rows_per_sub)])

    return k


def _make_sc_agg(NP, D, EP):
    n_idx_rows = EP // _CH
    rows_per_worker = n_idx_rows // (_NC * _NS)
    acc_rows_per_sub = NP // _NS
    mesh = plsc.VectorSubcoreMesh(core_axis_name="c", subcore_axis_name="s")

    @functools.partial(
        pl.kernel,
        mesh=mesh,
        out_type=jax.ShapeDtypeStruct((_NC * NP, D), jnp.float32),
        scratch_types=[
            pltpu.VMEM((rows_per_worker, _CH), jnp.int32),
            pltpu.VMEM((rows_per_worker, _CH), jnp.int32),
            pltpu.VMEM((_ZR, D), jnp.float32),
            pltpu.VMEM((_CH, D), jnp.float32),
            pltpu.VMEM((_CH, D), jnp.float32),
            pltpu.VMEM_SHARED((NP, D), jnp.float32),
            pltpu.VMEM_SHARED((NP, D), jnp.float32),
            pltpu.SemaphoreType.DMA,
            pltpu.SemaphoreType.DMA,
            pltpu.SemaphoreType.DMA,
            pltpu.SemaphoreType.DMA,
        ],
        compiler_params=pltpu.CompilerParams(use_tc_tiling_on_sc=False),
    )
    def k(g_hbm, src_hbm, dst_hbm, out_hbm, src_v, dst_v, zbuf, rbuf, rbuf1,
          gsh, acc, sem, sem1, ssem, ssem1):
        c = lax.axis_index("c")
        s = lax.axis_index("s")

        def zstore(i, _):
            r = i // (D // 16)
            col = (i % (D // 16)) * 16
            zbuf[r, pl.ds(col, 16)] = jnp.zeros((16,), jnp.float32)
            return 0
        lax.fori_loop(0, _ZR * D // 16, zstore, 0)

        row_base = s * acc_rows_per_sub
        wrow = (c * _NS + s) * rows_per_worker
        pltpu.async_copy(dst_hbm.at[pl.ds(wrow, rows_per_worker)], dst_v, ssem)
        pltpu.async_copy(src_hbm.at[pl.ds(wrow, rows_per_worker)], src_v,
                         ssem1)
        gstage = pltpu.make_async_copy(
            g_hbm.at[pl.ds(row_base, acc_rows_per_sub)],
            gsh.at[pl.ds(row_base, acc_rows_per_sub)], sem1)
        gstage.start()

        def zdesc(j):
            return pltpu.make_async_copy(
                zbuf, acc.at[pl.ds(row_base + j * _ZR, _ZR)], sem)
        def zcopy(j, _):
            zdesc(j).start()
            return 0
        lax.fori_loop(0, acc_rows_per_sub // _ZR, zcopy, 0)
        def zwait(j, _):
            zdesc(j).wait()
            return 0
        lax.fori_loop(0, acc_rows_per_sub // _ZR, zwait, 0)
        gstage.wait()
        pltpu.make_async_copy(
            dst_hbm.at[pl.ds(wrow, rows_per_worker)], dst_v, ssem).wait()
        pltpu.make_async_copy(
            src_hbm.at[pl.ds(wrow, rows_per_worker)], src_v, ssem1).wait()
        plsc.subcore_barrier()

        def _g(j, buf, sm):
            return pltpu.make_async_copy(gsh.at[src_v.at[j]], buf, sm)
        _g(0, rbuf, sem).start()
        if rows_per_worker > 1:
            _g(1, rbuf1, sem1).start()

        def step(j2, _):
            j = 2 * j2
            _g(j, rbuf, sem).wait()
            pltpu.sync_copy(rbuf, acc.at[dst_v.at[j]], add=True)
            @pl.when(j + 2 < rows_per_worker)
            def _():
                _g(j + 2, rbuf, sem).start()
            @pl.when(j + 1 < rows_per_worker)
            def _():
                _g(j + 1, rbuf1, sem1).wait()
                pltpu.sync_copy(rbuf1, acc.at[dst_v.at[j + 1]], add=True)
                @pl.when(j + 3 < rows_per_worker)
                def _():
                    _g(j + 3, rbuf1, sem1).start()
            return 0
        lax.fori_loop(0, (rows_per_worker + 1) // 2, step, 0)
        plsc.subcore_barrier()

        pltpu.sync_copy(acc.at[pl.ds(row_base, acc_rows_per_sub)],
                        out_hbm.at[pl.ds(c * NP + row_base, acc_rows_per_sub)])

    return k


def _make_sc_agg2(NP, D, EP):
    n_idx_rows = EP // _CH
    rows_per_worker = n_idx_rows // (_NC * _NS)
    acc_rows_per_sub = NP // _NS
    mesh = plsc.VectorSubcoreMesh(core_axis_name="c", subcore_axis_name="s")

    @functools.partial(
        pl.kernel,
        mesh=mesh,
        out_type=[jax.ShapeDtypeStruct((_NC * NP, D), jnp.float32),
                  jax.ShapeDtypeStruct((_NC * NP, D), jnp.float32)],
        scratch_types=[
            pltpu.VMEM((rows_per_worker, _CH), jnp.int32),
            pltpu.VMEM((rows_per_worker, _CH), jnp.int32),
            pltpu.VMEM((_ZR, D), jnp.float32),
            pltpu.VMEM((_CH, D), jnp.float32),
            pltpu.VMEM((_CH, D), jnp.float32),
            pltpu.VMEM_SHARED((NP, D), jnp.float32),
            pltpu.VMEM_SHARED((NP, D), jnp.float32),
            pltpu.SemaphoreType.DMA,
            pltpu.SemaphoreType.DMA,
            pltpu.SemaphoreType.DMA,
            pltpu.SemaphoreType.DMA,
        ],
        compiler_params=pltpu.CompilerParams(use_tc_tiling_on_sc=False),
    )
    def k(gl_hbm, gr_hbm, src_hbm, dst_hbm, outl_hbm, outr_hbm, src_v, dst_v,
          zbuf, rbuf, rbuf1, gsh, acc, sem, sem1, ssem, ssem1):
        c = lax.axis_index("c")
        s = lax.axis_index("s")

        def zstore(i, _):
            r = i // (D // 16)
            col = (i % (D // 16)) * 16
            zbuf[r, pl.ds(col, 16)] = jnp.zeros((16,), jnp.float32)
            return 0
        lax.fori_loop(0, _ZR * D // 16, zstore, 0)

        row_base = s * acc_rows_per_sub
        wrow = (c * _NS + s) * rows_per_worker
        pltpu.async_copy(dst_hbm.at[pl.ds(wrow, rows_per_worker)], dst_v, ssem)
        pltpu.async_copy(src_hbm.at[pl.ds(wrow, rows_per_worker)], src_v,
                         ssem1)

        def gstage(g_hbm):
            return pltpu.make_async_copy(
                g_hbm.at[pl.ds(row_base, acc_rows_per_sub)],
                gsh.at[pl.ds(row_base, acc_rows_per_sub)], sem1)

        def zdesc(j):
            return pltpu.make_async_copy(
                zbuf, acc.at[pl.ds(row_base + j * _ZR, _ZR)], sem)

        def zero_acc():
            def zcopy(j, _):
                zdesc(j).start()
                return 0
            lax.fori_loop(0, acc_rows_per_sub // _ZR, zcopy, 0)
            def zwait(j, _):
                zdesc(j).wait()
                return 0
            lax.fori_loop(0, acc_rows_per_sub // _ZR, zwait, 0)

        def edge_pipeline():
            def _g(j, buf, sm):
                return pltpu.make_async_copy(gsh.at[src_v.at[j]], buf, sm)
            _g(0, rbuf, sem).start()
            if rows_per_worker > 1:
                _g(1, rbuf1, sem1).start()

            def step(j2, _):
                j = 2 * j2
                _g(j, rbuf, sem).wait()
                pltpu.sync_copy(rbuf, acc.at[dst_v.at[j]], add=True)
                @pl.when(j + 2 < rows_per_worker)
                def _():
                    _g(j + 2, rbuf, sem).start()
                @pl.when(j + 1 < rows_per_worker)
                def _():
                    _g(j + 1, rbuf1, sem1).wait()
                    pltpu.sync_copy(rbuf1, acc.at[dst_v.at[j + 1]], add=True)
                    @pl.when(j + 3 < rows_per_worker)
                    def _():
                        _g(j + 3, rbuf1, sem1).start()
                return 0
            lax.fori_loop(0, (rows_per_worker + 1) // 2, step, 0)

        def out_desc(out_hbm, sm):
            return pltpu.make_async_copy(
                acc.at[pl.ds(row_base, acc_rows_per_sub)],
                out_hbm.at[pl.ds(c * NP + row_base, acc_rows_per_sub)], sm)

        stage = gstage(gl_hbm)
        stage.start()
        zero_acc()
        stage.wait()
        pltpu.make_async_copy(
            dst_hbm.at[pl.ds(wrow, rows_per_worker)], dst_v, ssem).wait()
        pltpu.make_async_copy(
            src_hbm.at[pl.ds(wrow, rows_per_worker)], src_v, ssem1).wait()
        plsc.subcore_barrier()
        edge_pipeline()
        plsc.subcore_barrier()
        outl = out_desc(outl_hbm, ssem)
        outl.start()

        stage = gstage(gr_hbm)
        stage.start()
        outl.wait()
        zero_acc()
        stage.wait()
        plsc.subcore_barrier()
        edge_pipeline()
        plsc.subcore_barrier()
        out_desc(outr_hbm, ssem).start()
        out_desc(outr_hbm, ssem).wait()

    return k


def _dis(dega_ref, degb_ref):
    deg = dega_ref[...][:, :1] + degb_ref[...][:, :1] + 1.0
    return lax.rsqrt(deg)


def _half_specs(d, nblk):
    return [pl.BlockSpec((_BM, d), lambda i: (i, 0)),
            pl.BlockSpec((_BM, d), lambda i, nb=nblk: (i + nb, 0))]


def _mm1_body(dega_ref, degb_ref, x_ref, w_ref, o1_ref, o2_ref):
    dis = _dis(dega_ref, degb_ref)
    h = w_ref.shape[1] // 2
    res = jnp.dot(x_ref[...], w_ref[...],
                  preferred_element_type=jnp.float32) * dis
    o1_ref[...] = res[:, :h]
    o2_ref[...] = res[:, h:]


def _mid_body(dega_ref, degb_ref, sla_ref, slb_ref, sra_ref, srb_ref,
              g1l_ref, g1r_ref, b1_ref, w2_ref, o_ref):
    dis = _dis(dega_ref, degb_ref)
    h = g1l_ref.shape[1]
    ul = dis * (sla_ref[...] + slb_ref[...] + g1l_ref[...]) + b1_ref[:, :h]
    ur = dis * (sra_ref[...] + srb_ref[...] + g1r_ref[...]) + b1_ref[:, h:]
    u = jnp.maximum(jnp.concatenate([ul, ur], axis=1), 0.0)
    o_ref[...] = jnp.dot(u, w2_ref[...], preferred_element_type=jnp.float32) * dis


def _fin_body(dega_ref, degb_ref, s2a_ref, s2b_ref, g2_ref, b2_ref, o_ref):
    dis = _dis(dega_ref, degb_ref)
    o_ref[...] = dis * (s2a_ref[...] + s2b_ref[...] + g2_ref[...]) + b2_ref[...]


def _row_spec(d):
    return pl.BlockSpec((_BM, d), lambda i: (i, 0))


def _full_spec(r, c):
    return pl.BlockSpec((r, c), lambda i: (0, 0))


def kernel(x, edge_index, W1, b1, W2, b2):
    N, F = x.shape
    E = edge_index.shape[1]
    D1 = W1.shape[1]
    D2 = W2.shape[1]
    DH = D1 // 2
    NP = _round_up(N + 1, _NS * _ZR)
    EP = _round_up(E, _NC * _NS * _CH)

    pad = jnp.full((EP - E,), N, jnp.int32)
    src = jnp.concatenate([edge_index[0], pad]).reshape(EP // _CH, _CH)
    dst = jnp.concatenate([edge_index[1], pad]).reshape(EP // _CH, _CH)

    degp = _make_sc_deg(NP, EP)(dst)

    nblk = NP // _BM
    grid = (nblk,)
    deg_specs = _half_specs(_DEGW, nblk)

    g1l, g1r = pl.pallas_call(
        _mm1_body,
        grid=grid,
        in_specs=deg_specs + [_row_spec(F), _full_spec(F, D1)],
        out_specs=[_row_spec(DH), _row_spec(DH)],
        out_shape=[jax.ShapeDtypeStruct((NP, DH), jnp.float32),
                   jax.ShapeDtypeStruct((NP, DH), jnp.float32)],
    )(degp, degp, x, W1)

    s1l, s1r = _make_sc_agg2(NP, DH, EP)(g1l, g1r, src, dst)

    g2 = pl.pallas_call(
        _mid_body,
        grid=grid,
        in_specs=(deg_specs + _half_specs(DH, nblk) + _half_specs(DH, nblk)
                  + [_row_spec(DH), _row_spec(DH), _full_spec(1, D1),
                     _full_spec(D1, D2)]),
        out_specs=_row_spec(D2),
        out_shape=jax.ShapeDtypeStruct((NP, D2), jnp.float32),
    )(degp, degp, s1l, s1l, s1r, s1r, g1l, g1r, b1.reshape(1, D1), W2)

    s2 = _make_sc_agg(NP, D2, EP)(g2, src, dst)

    out = pl.pallas_call(
        _fin_body,
        grid=grid,
        in_specs=deg_specs + _half_specs(D2, nblk) + [_row_spec(D2),
                  _full_spec(1, D2)],
        out_specs=_row_spec(D2),
        out_shape=jax.ShapeDtypeStruct((N, D2), jnp.float32),
    )(degp, degp, s2, s2, g2, b2.reshape(1, D2))

    return out
